# baseline (speedup 1.0000x reference)
import functools

import jax
import jax.numpy as jnp
from jax import lax
from jax.experimental import pallas as pl
from jax.experimental.pallas import tpu as pltpu
from jax.experimental.pallas import tpu_sc as plsc

N = 10000
K = 32
DF = 128
U = 128

CH = 128
NBUF = 4
T = 80
NP = T * CH
S = T // NBUF


def _matmul_body(x_ref, w_ref, o_ref):
    o_ref[...] = jnp.dot(x_ref[...], w_ref[...],
                         preferred_element_type=jnp.float32)


def _matmul(x, w):
    B = 2000
    return pl.pallas_call(
        _matmul_body,
        grid=(N // B,),
        in_specs=[
            pl.BlockSpec((B, DF), lambda i: (i, 0)),
            pl.BlockSpec((DF, U), lambda i: (0, 0)),
        ],
        out_specs=pl.BlockSpec((B, U), lambda i: (i, 0)),
        out_shape=jax.ShapeDtypeStruct((N, U), jnp.float32),
    )(x, w)


def _sc_gather(table, idx):
    info = plsc.get_sparse_core_info()
    nc = info.num_cores
    mesh = plsc.VectorSubcoreMesh(core_axis_name="c", subcore_axis_name="s")
    LAG = 3

    @functools.partial(
        pl.kernel,
        mesh=mesh,
        out_type=jax.ShapeDtypeStruct((K, NP, U), jnp.float32),
        scratch_types=[
            pltpu.VMEM((T, CH), jnp.int32),
            pltpu.VMEM((NBUF, CH, U), jnp.float32),
            pltpu.SemaphoreType.DMA,
            pltpu.SemaphoreType.DMA((NBUF,)),
            pltpu.SemaphoreType.DMA((NBUF,)),
        ],
    )
    def gk(table_hbm, idx_hbm, out_hbm, idx_v, rows_v, isem, gsem, wsem):
        w = lax.axis_index("s") * nc + lax.axis_index("c")

        pltpu.make_async_copy(idx_hbm.at[w], idx_v, isem).start()
        pltpu.make_async_copy(idx_hbm.at[w], idx_v, isem).wait()

        def gather(t):
            return pltpu.make_async_copy(
                table_hbm.at[idx_v.at[t]], rows_v.at[t % NBUF],
                gsem.at[t % NBUF])

        def wback(t):
            return pltpu.make_async_copy(
                rows_v.at[t % NBUF], out_hbm.at[w, pl.ds(t * CH, CH)],
                wsem.at[t % NBUF])

        for t in range(T):
            gather(t).start()
            gather(t).wait()
            wback(t).start()
            wback(t).wait()

    return gk(table, idx)


def _batcher_pairs(n):
    pairs = []
    p = 1
    while p < n:
        k = p
        while k >= 1:
            for j in range(k % p, n - k, 2 * k):
                for i in range(min(k, n - j - k)):
                    if (i + j) // (2 * p) == (i + j + k) // (2 * p):
                        pairs.append((i + j, i + j + k))
            k //= 2
        p *= 2
    return pairs


_PAIRS16 = _batcher_pairs(16)


def _sort16(vals):
    vals = list(vals)
    for a, b in _PAIRS16:
        lo = jnp.minimum(vals[a], vals[b])
        hi = jnp.maximum(vals[a], vals[b])
        vals[a], vals[b] = lo, hi
    return vals


def _median32(vals):
    a = _sort16(vals[:16])
    b = _sort16(vals[16:])
    lo = [jnp.minimum(a[i], b[15 - i]) for i in range(16)]
    hi = [jnp.maximum(a[i], b[15 - i]) for i in range(16)]
    mx = functools.reduce(jnp.maximum, lo)
    mn = functools.reduce(jnp.minimum, hi)
    return (mx + mn) * 0.5


def _median_body(msg_ref, o_ref):
    vals = [msg_ref[k] for k in range(K)]
    o_ref[...] = _median32(vals)


def _median(msg):
    B = 200
    return pl.pallas_call(
        _median_body,
        grid=(N // B,),
        in_specs=[pl.BlockSpec((K, B, U), lambda i: (0, i, 0))],
        out_specs=pl.BlockSpec((B, U), lambda i: (i, 0)),
        out_shape=jax.ShapeDtypeStruct((N, U), jnp.float32),
    )(msg)


def kernel(x, neighbors, kernel):
    w = kernel
    h = _matmul(x, w)
    idxt = neighbors.astype(jnp.int32).T
    idxt = jnp.pad(idxt, ((0, 0), (0, NP - N))).reshape(K, T, CH)
    msg = _sc_gather(h, idxt)
    return _median(msg)


# speedup vs baseline: 1.1031x; 1.1031x over previous
import functools

import jax
import jax.numpy as jnp
from jax import lax
from jax.experimental import pallas as pl
from jax.experimental.pallas import tpu as pltpu
from jax.experimental.pallas import tpu_sc as plsc

N = 10000
K = 32
DF = 128
U = 128

CH = 128
NBUF = 4
T = 80
NP = T * CH
S = T // NBUF


def _matmul_body(x_ref, w_ref, o_ref):
    o_ref[...] = jnp.dot(x_ref[...], w_ref[...],
                         preferred_element_type=jnp.float32)


def _matmul(x, w):
    B = 2000
    return pl.pallas_call(
        _matmul_body,
        grid=(N // B,),
        in_specs=[
            pl.BlockSpec((B, DF), lambda i: (i, 0)),
            pl.BlockSpec((DF, U), lambda i: (0, 0)),
        ],
        out_specs=pl.BlockSpec((B, U), lambda i: (i, 0)),
        out_shape=jax.ShapeDtypeStruct((N, U), jnp.float32),
    )(x, w)


def _sc_gather(table, idx):
    info = plsc.get_sparse_core_info()
    nc = info.num_cores
    mesh = plsc.VectorSubcoreMesh(core_axis_name="c", subcore_axis_name="s")
    LAG = 3
    NIB = 6

    @functools.partial(
        pl.kernel,
        mesh=mesh,
        out_type=jax.ShapeDtypeStruct((K, NP, U), jnp.float32),
        scratch_types=(
            [pltpu.VMEM((CH,), jnp.int32)] * NIB
            + [pltpu.VMEM((NBUF, CH, U), jnp.float32),
               pltpu.SemaphoreType.DMA((NIB,)),
               pltpu.SemaphoreType.DMA((NBUF,)),
               pltpu.SemaphoreType.DMA((NBUF,))]
        ),
    )
    def gk(table_hbm, idx_hbm, out_hbm, *rest):
        ibufs = rest[:NIB]
        rows_v, isem, gsem, wsem = rest[NIB:]
        w = lax.axis_index("s") * nc + lax.axis_index("c")

        def idx_cp(t):
            return pltpu.make_async_copy(
                idx_hbm.at[w, t], ibufs[t % NIB], isem.at[t % NIB])

        def gather(t):
            return pltpu.make_async_copy(
                table_hbm.at[ibufs[t % NIB]], rows_v.at[t % NBUF],
                gsem.at[t % NBUF])

        def wback(t):
            return pltpu.make_async_copy(
                rows_v.at[t % NBUF], out_hbm.at[w, pl.ds(t * CH, CH)],
                wsem.at[t % NBUF])

        idx_cp(0).start()
        idx_cp(1).start()
        for t in range(T + LAG):
            if t < T:
                if t >= NBUF:
                    wback(t - NBUF).wait()
                idx_cp(t).wait()
                gather(t).start()
            if t + 2 < T:
                idx_cp(t + 2).start()
            u = t - LAG
            if u >= 0:
                gather(u).wait()
                wback(u).start()
        for u in range(T - NBUF, T):
            wback(u).wait()

    return gk(table, idx)


def _batcher_pairs(n):
    pairs = []
    p = 1
    while p < n:
        k = p
        while k >= 1:
            for j in range(k % p, n - k, 2 * k):
                for i in range(min(k, n - j - k)):
                    if (i + j) // (2 * p) == (i + j + k) // (2 * p):
                        pairs.append((i + j, i + j + k))
            k //= 2
        p *= 2
    return pairs


_PAIRS16 = _batcher_pairs(16)


def _sort16(vals):
    vals = list(vals)
    for a, b in _PAIRS16:
        lo = jnp.minimum(vals[a], vals[b])
        hi = jnp.maximum(vals[a], vals[b])
        vals[a], vals[b] = lo, hi
    return vals


def _median32(vals):
    a = _sort16(vals[:16])
    b = _sort16(vals[16:])
    lo = [jnp.minimum(a[i], b[15 - i]) for i in range(16)]
    hi = [jnp.maximum(a[i], b[15 - i]) for i in range(16)]
    mx = functools.reduce(jnp.maximum, lo)
    mn = functools.reduce(jnp.minimum, hi)
    return (mx + mn) * 0.5


def _median_body(msg_ref, o_ref):
    vals = [msg_ref[k] for k in range(K)]
    o_ref[...] = _median32(vals)


def _median(msg):
    B = 200
    return pl.pallas_call(
        _median_body,
        grid=(N // B,),
        in_specs=[pl.BlockSpec((K, B, U), lambda i: (0, i, 0))],
        out_specs=pl.BlockSpec((B, U), lambda i: (i, 0)),
        out_shape=jax.ShapeDtypeStruct((N, U), jnp.float32),
    )(msg)


def kernel(x, neighbors, kernel):
    w = kernel
    h = _matmul(x, w)
    idxt = neighbors.astype(jnp.int32).T
    idxt = jnp.pad(idxt, ((0, 0), (0, NP - N))).reshape(K, T, CH)
    msg = _sc_gather(h, idxt)
    return _median(msg)
